# baseline (speedup 1.0000x reference)
import jax
import jax.numpy as jnp
from jax import lax
from jax.experimental import pallas as pl
from jax.experimental.pallas import tpu as pltpu


def _pos_kernel(row_ref, col_ref, out_ref, scratch, sem):
    H, W, D = 32, 32, 256
    HW = H * W
    col = col_ref[0:W, :]
    row = row_ref[0:H, :]
    x_tile = jnp.concatenate([col] * H, axis=0)
    j = lax.broadcasted_iota(jnp.int32, (HW, H), 0)
    hsel = lax.broadcasted_iota(jnp.int32, (HW, H), 1)
    rep = (j // W == hsel).astype(jnp.float32)
    dn = (((1,), (0,)), ((), ()))
    y_tile = lax.dot_general(rep, row, dn, preferred_element_type=jnp.float32)
    tile = jnp.concatenate([x_tile, y_tile], axis=1)
    scratch[0] = tile
    scratch[1] = tile
    copies = [
        pltpu.make_async_copy(scratch, out_ref.at[p], sem.at[p])
        for p in range(out_ref.shape[0])
    ]
    for c in copies:
        c.start()
    for c in copies:
        c.wait()


def kernel(row_embeddings, column_embeddings, x):
    batch, _, height, width = x.shape
    D = row_embeddings.shape[1]
    C = 2 * D
    HW = height * width
    out = pl.pallas_call(
        _pos_kernel,
        in_specs=[
            pl.BlockSpec(memory_space=pltpu.MemorySpace.VMEM),
            pl.BlockSpec(memory_space=pltpu.MemorySpace.VMEM),
        ],
        out_specs=pl.BlockSpec(memory_space=pltpu.MemorySpace.HBM),
        out_shape=jax.ShapeDtypeStruct((batch // 2, 2, HW, C), jnp.float32),
        scratch_shapes=[
            pltpu.VMEM((2, HW, C), jnp.float32),
            pltpu.SemaphoreType.DMA((batch // 2,)),
        ],
    )(row_embeddings, column_embeddings)
    return out.reshape(batch, height, width, C).transpose(0, 3, 1, 2)
